# baseline (speedup 1.0000x reference)
import functools

import jax
import jax.numpy as jnp
from jax import lax
from jax.experimental import pallas as pl
from jax.experimental.pallas import tpu as pltpu
from jax.experimental.pallas import tpu_sc as plsc

_B, _L, _D, _H = 64, 200, 768, 8
_R = 2 * _B * _L


def _gather_sc(idx, table, nrows, ch):
    info = plsc.get_sparse_core_info()
    nw = info.num_cores * info.num_subcores
    r_per_w = nrows // nw
    nchunk = r_per_w // ch

    mesh = plsc.VectorSubcoreMesh(core_axis_name="c", subcore_axis_name="s")

    @functools.partial(
        pl.kernel,
        mesh=mesh,
        out_type=jax.ShapeDtypeStruct((nrows, _D), jnp.float32),
        scratch_types=[
            pltpu.VMEM((ch,), jnp.int32),
            pltpu.VMEM((ch, _D), jnp.float32),
            pltpu.SemaphoreType.DMA,
        ],
    )
    def gather_kernel(idx_hbm, table_hbm, out_hbm, idx_v, rows_v, sem):
        wid = lax.axis_index("s") * info.num_cores + lax.axis_index("c")
        base = wid * r_per_w

        def body(i, carry):
            off = base + i * ch
            pltpu.sync_copy(idx_hbm.at[pl.ds(off, ch)], idx_v)
            pltpu.async_copy(table_hbm.at[idx_v], rows_v, sem).wait()
            pltpu.sync_copy(rows_v, out_hbm.at[pl.ds(off, ch)])
            return carry

        lax.fori_loop(0, nchunk, body, 0)

    return gather_kernel(idx, table)


def _w2_packed_perm():
    import numpy as np
    pos = np.arange(_D)
    g, r = pos // 32, pos % 32
    old = 32 * g + (r // 2) + 16 * (r % 2)
    return jnp.asarray(old, dtype=jnp.int32)


_NB = 4


def _attn_body(gs_ref, go_ref, sv_ref, ov_ref, w2_ref, b2_ref, w1s_ref,
               w1o_ref, b1_ref, oh_ref, logits_ref, loss_ref, satt_ref,
               oatt_ref):
    step = pl.program_id(0)

    def side(feat, vals):
        qT = lax.dot_general(w2_ref[...], feat.astype(jnp.bfloat16),
                             (((1,), (1,)), ((), ())),
                             preferred_element_type=jnp.float32)
        qT = jnp.maximum(qT + b2_ref[...], 0.0)
        m = jnp.max(qT, axis=1, keepdims=True)
        e = jnp.exp(qT - m)
        s = jnp.sum(e, axis=1, keepdims=True)
        attT = jnp.mean(e / s, axis=0, keepdims=True)
        dep = lax.dot_general(attT, vals, (((1,), (0,)), ((), ())),
                              preferred_element_type=jnp.float32)
        return attT, dep

    nll_sum = jnp.zeros((1, 1), jnp.float32)
    for i in range(_NB):
        satt, dep_s = side(gs_ref[i], sv_ref[i])
        oatt, dep_o = side(go_ref[i], ov_ref[i])
        satt_ref[i] = satt
        oatt_ref[i] = oatt

        logits2 = (
            lax.dot_general(dep_s, w1s_ref[...], (((1,), (0,)), ((), ())),
                            preferred_element_type=jnp.float32)
            + lax.dot_general(dep_o, w1o_ref[...], (((1,), (0,)), ((), ())),
                              preferred_element_type=jnp.float32)
            + b1_ref[...]
        )
        logits_ref[i] = logits2

        l0 = logits2[0, 0]
        l1 = logits2[0, 1]
        m2 = jnp.maximum(l0, l1)
        logz = m2 + jnp.log(jnp.exp(l0 - m2) + jnp.exp(l1 - m2))
        picked = jnp.sum(logits2 * oh_ref[i])
        nll_sum = nll_sum + jnp.reshape((logz - picked) * (1.0 / _B), (1, 1))

    @pl.when(step == 0)
    def _():
        loss_ref[...] = jnp.zeros((1, 1), jnp.float32)

    loss_ref[...] = loss_ref[...] + nll_sum


_NCK = 2
_BC = _B // _NCK


def _attention_tc(gf, subj_vals, obj_vals, w2p, b2, W1, b1, oh3, cidx):
    w1sT = W1[:, :_D].T
    w1oT = W1[:, _D:].T
    b2c = b2.reshape(_H, 1)
    b1r = b1.reshape(1, 2)

    nsteps = _BC // _NB
    voff = cidx * nsteps
    outs = pl.pallas_call(
        _attn_body,
        grid=(nsteps,),
        in_specs=[
            pl.BlockSpec((_NB, _L, _D), lambda b: (b, 0, 0)),
            pl.BlockSpec((_NB, _L, _D),
                         lambda b, n=nsteps: (n + b, 0, 0)),
            pl.BlockSpec((_NB, _L, _D),
                         lambda b, v=voff: (v + b, 0, 0)),
            pl.BlockSpec((_NB, _L, _D),
                         lambda b, v=voff: (v + b, 0, 0)),
            pl.BlockSpec((_H, _D), lambda b: (0, 0)),
            pl.BlockSpec((_H, 1), lambda b: (0, 0)),
            pl.BlockSpec((_D, 2), lambda b: (0, 0)),
            pl.BlockSpec((_D, 2), lambda b: (0, 0)),
            pl.BlockSpec((1, 2), lambda b: (0, 0)),
            pl.BlockSpec((_NB, 1, 2), lambda b: (b, 0, 0)),
        ],
        out_specs=[
            pl.BlockSpec((_NB, 1, 2), lambda b: (b, 0, 0)),
            pl.BlockSpec((1, 1), lambda b: (0, 0)),
            pl.BlockSpec((_NB, 1, _L), lambda b: (b, 0, 0)),
            pl.BlockSpec((_NB, 1, _L), lambda b: (b, 0, 0)),
        ],
        out_shape=[
            jax.ShapeDtypeStruct((_BC, 1, 2), jnp.float32),
            jax.ShapeDtypeStruct((1, 1), jnp.float32),
            jax.ShapeDtypeStruct((_BC, 1, _L), jnp.float32),
            jax.ShapeDtypeStruct((_BC, 1, _L), jnp.float32),
        ],
    )(gf, gf, subj_vals, obj_vals, w2p, b2c, w1sT, w1oT, b1r, oh3)

    logits3, loss11, satt3, oatt3 = outs
    return logits3[:, 0, :], loss11, satt3[:, 0, :], oatt3[:, 0, :]


def kernel(target_relation_id, subj_property_ids, obj_property_ids,
           subj_value_embeds, obj_value_embeds, label_ids,
           property_table, W2, b2, W1, b1):
    onehot = jax.nn.one_hot(label_ids, 2, dtype=jnp.float32)
    oh3 = onehot.reshape(_B, 1, 2)
    w2p = W2.astype(jnp.bfloat16)
    parts = []
    for c in range(_NCK):
        sid = subj_property_ids[c * _BC:(c + 1) * _BC].reshape(-1)
        oid = obj_property_ids[c * _BC:(c + 1) * _BC].reshape(-1)
        idx = jnp.concatenate([sid, oid]).astype(jnp.int32)
        g = _gather_sc(idx, property_table, 2 * _BC * _L, ch=80)
        gf = g.reshape(2 * _BC, _L, _D)
        parts.append(_attention_tc(gf, subj_value_embeds, obj_value_embeds,
                                   w2p, b2, W1, b1,
                                   oh3[c * _BC:(c + 1) * _BC], c))
    logits = jnp.concatenate([p[0] for p in parts], axis=0)
    loss = sum(p[1] for p in parts).reshape(())
    satt = jnp.concatenate([p[2] for p in parts], axis=0)
    oatt = jnp.concatenate([p[3] for p in parts], axis=0)
    return logits, loss, satt, oatt


# speedup vs baseline: 1.0142x; 1.0142x over previous
import functools

import jax
import jax.numpy as jnp
from jax import lax
from jax.experimental import pallas as pl
from jax.experimental.pallas import tpu as pltpu
from jax.experimental.pallas import tpu_sc as plsc

_B, _L, _D, _H = 64, 200, 768, 8
_R = 2 * _B * _L


def _gather_sc(idx, table, nrows, ch):
    info = plsc.get_sparse_core_info()
    nw = info.num_cores * info.num_subcores
    r_per_w = nrows // nw
    nchunk = r_per_w // ch

    mesh = plsc.VectorSubcoreMesh(core_axis_name="c", subcore_axis_name="s")

    @functools.partial(
        pl.kernel,
        mesh=mesh,
        out_type=jax.ShapeDtypeStruct((nrows, _D), jnp.float32),
        scratch_types=[
            pltpu.VMEM((ch,), jnp.int32),
            pltpu.VMEM((ch, _D), jnp.float32),
            pltpu.SemaphoreType.DMA,
        ],
    )
    def gather_kernel(idx_hbm, table_hbm, out_hbm, idx_v, rows_v, sem):
        wid = lax.axis_index("s") * info.num_cores + lax.axis_index("c")
        base = wid * r_per_w

        def body(i, carry):
            off = base + i * ch
            pltpu.sync_copy(idx_hbm.at[pl.ds(off, ch)], idx_v)
            pltpu.async_copy(table_hbm.at[idx_v], rows_v, sem).wait()
            pltpu.sync_copy(rows_v, out_hbm.at[pl.ds(off, ch)])
            return carry

        lax.fori_loop(0, nchunk, body, 0)

    return gather_kernel(idx, table)


def _w2_packed_perm():
    import numpy as np
    pos = np.arange(_D)
    g, r = pos // 32, pos % 32
    old = 32 * g + (r // 2) + 16 * (r % 2)
    return jnp.asarray(old, dtype=jnp.int32)


_NB = 4


def _attn_body(gs_ref, go_ref, sv_ref, ov_ref, w2_ref, b2_ref, w1s_ref,
               w1o_ref, b1_ref, oh_ref, logits_ref, loss_ref, satt_ref,
               oatt_ref):
    step = pl.program_id(0)

    def side(feat, vals):
        qT = lax.dot_general(w2_ref[...], feat.astype(jnp.bfloat16),
                             (((1,), (1,)), ((), ())),
                             preferred_element_type=jnp.float32)
        qT = jnp.maximum(qT + b2_ref[...], 0.0)
        m = jnp.max(qT, axis=1, keepdims=True)
        e = jnp.exp(qT - m)
        s = jnp.sum(e, axis=1, keepdims=True)
        attT = jnp.mean(e / s, axis=0, keepdims=True)
        dep = lax.dot_general(attT, vals, (((1,), (0,)), ((), ())),
                              preferred_element_type=jnp.float32)
        return attT, dep

    nll_sum = jnp.zeros((1, 1), jnp.float32)
    for i in range(_NB):
        satt, dep_s = side(gs_ref[i], sv_ref[i])
        oatt, dep_o = side(go_ref[i], ov_ref[i])
        satt_ref[i] = satt
        oatt_ref[i] = oatt

        logits2 = (
            lax.dot_general(dep_s, w1s_ref[...], (((1,), (0,)), ((), ())),
                            preferred_element_type=jnp.float32)
            + lax.dot_general(dep_o, w1o_ref[...], (((1,), (0,)), ((), ())),
                              preferred_element_type=jnp.float32)
            + b1_ref[...]
        )
        logits_ref[i] = logits2

        l0 = logits2[0, 0]
        l1 = logits2[0, 1]
        m2 = jnp.maximum(l0, l1)
        logz = m2 + jnp.log(jnp.exp(l0 - m2) + jnp.exp(l1 - m2))
        picked = jnp.sum(logits2 * oh_ref[i])
        nll_sum = nll_sum + jnp.reshape((logz - picked) * (1.0 / _B), (1, 1))

    @pl.when(step == 0)
    def _():
        loss_ref[...] = jnp.zeros((1, 1), jnp.float32)

    loss_ref[...] = loss_ref[...] + nll_sum


_NCK = 4
_BC = _B // _NCK


def _attention_tc(gf, subj_vals, obj_vals, w2p, b2, W1, b1, oh3, cidx):
    w1sT = W1[:, :_D].T
    w1oT = W1[:, _D:].T
    b2c = b2.reshape(_H, 1)
    b1r = b1.reshape(1, 2)

    nsteps = _BC // _NB
    voff = cidx * nsteps
    outs = pl.pallas_call(
        _attn_body,
        grid=(nsteps,),
        in_specs=[
            pl.BlockSpec((_NB, _L, _D), lambda b: (b, 0, 0)),
            pl.BlockSpec((_NB, _L, _D),
                         lambda b, n=nsteps: (n + b, 0, 0)),
            pl.BlockSpec((_NB, _L, _D),
                         lambda b, v=voff: (v + b, 0, 0)),
            pl.BlockSpec((_NB, _L, _D),
                         lambda b, v=voff: (v + b, 0, 0)),
            pl.BlockSpec((_H, _D), lambda b: (0, 0)),
            pl.BlockSpec((_H, 1), lambda b: (0, 0)),
            pl.BlockSpec((_D, 2), lambda b: (0, 0)),
            pl.BlockSpec((_D, 2), lambda b: (0, 0)),
            pl.BlockSpec((1, 2), lambda b: (0, 0)),
            pl.BlockSpec((_NB, 1, 2), lambda b: (b, 0, 0)),
        ],
        out_specs=[
            pl.BlockSpec((_NB, 1, 2), lambda b: (b, 0, 0)),
            pl.BlockSpec((1, 1), lambda b: (0, 0)),
            pl.BlockSpec((_NB, 1, _L), lambda b: (b, 0, 0)),
            pl.BlockSpec((_NB, 1, _L), lambda b: (b, 0, 0)),
        ],
        out_shape=[
            jax.ShapeDtypeStruct((_BC, 1, 2), jnp.float32),
            jax.ShapeDtypeStruct((1, 1), jnp.float32),
            jax.ShapeDtypeStruct((_BC, 1, _L), jnp.float32),
            jax.ShapeDtypeStruct((_BC, 1, _L), jnp.float32),
        ],
    )(gf, gf, subj_vals, obj_vals, w2p, b2c, w1sT, w1oT, b1r, oh3)

    logits3, loss11, satt3, oatt3 = outs
    return logits3[:, 0, :], loss11, satt3[:, 0, :], oatt3[:, 0, :]


def kernel(target_relation_id, subj_property_ids, obj_property_ids,
           subj_value_embeds, obj_value_embeds, label_ids,
           property_table, W2, b2, W1, b1):
    onehot = jax.nn.one_hot(label_ids, 2, dtype=jnp.float32)
    oh3 = onehot.reshape(_B, 1, 2)
    w2p = W2.astype(jnp.bfloat16)
    parts = []
    for c in range(_NCK):
        sid = subj_property_ids[c * _BC:(c + 1) * _BC].reshape(-1)
        oid = obj_property_ids[c * _BC:(c + 1) * _BC].reshape(-1)
        idx = jnp.concatenate([sid, oid]).astype(jnp.int32)
        g = _gather_sc(idx, property_table, 2 * _BC * _L, ch=40)
        gf = g.reshape(2 * _BC, _L, _D)
        parts.append(_attention_tc(gf, subj_value_embeds, obj_value_embeds,
                                   w2p, b2, W1, b1,
                                   oh3[c * _BC:(c + 1) * _BC], c))
    logits = jnp.concatenate([p[0] for p in parts], axis=0)
    loss = sum(p[1] for p in parts).reshape(())
    satt = jnp.concatenate([p[2] for p in parts], axis=0)
    oatt = jnp.concatenate([p[3] for p in parts], axis=0)
    return logits, loss, satt, oatt
